# baseline (speedup 1.0000x reference)
import jax
import jax.numpy as jnp
from jax import lax
from jax.experimental import pallas as pl
from jax.experimental.pallas import tpu as pltpu
from jax.experimental.pallas import tpu_sc as plsc

_NUM_NODES = 1024
_INPUT_SIZE = 512
_OUTPUT_SIZE = 128
_FAN_IN = 16
_BATCH = 2048
_NC = 2
_NS = 16
_L = 16
_NW = _NC * _NS
_BPW = _BATCH // _NW
_NG = _BPW // _L
_CH = 128
_CHW = _CH * _FAN_IN
_NCHK = _NUM_NODES // _CH
_SLOTS = _INPUT_SIZE + _NUM_NODES


def _sc_body(xw_ref, idx_ref, w_ref, out_ref, a_ref, idx_v, w_v,
             sem_x, sem_i, sem_w):
    c = lax.axis_index("c")
    s = lax.axis_index("s")
    wid = s * _NC + c

    cp_x = pltpu.async_copy(xw_ref.at[wid],
                            a_ref.at[pl.ds(0, _INPUT_SIZE * _BPW)], sem_x)
    prev = (
        pltpu.async_copy(idx_ref.at[pl.ds(0, _CHW)],
                         idx_v.at[pl.ds(0, _CHW)], sem_i),
        pltpu.async_copy(w_ref.at[pl.ds(0, _CHW)],
                         w_v.at[pl.ds(0, _CHW)], sem_w),
    )
    cp_x.wait()

    lane = lax.iota(jnp.int32, _L)
    cols = [lane + (_L * g) for g in range(_NG)]
    gdn = lax.GatherDimensionNumbers(
        offset_dims=(), collapsed_slice_dims=(0,), start_index_map=(0,))

    def lane_bcast(vec, j):
        jidx = jnp.full((_L, 1), j, jnp.int32)
        return lax.gather(vec, jidx, gdn, slice_sizes=(1,),
                          mode=lax.GatherScatterMode.PROMISE_IN_BOUNDS)

    for ck in range(_NCHK):
        bf = ck % 2
        prev[0].wait()
        prev[1].wait()
        if ck + 1 < _NCHK:
            nbf = (ck + 1) % 2
            prev = (
                pltpu.async_copy(idx_ref.at[pl.ds((ck + 1) * _CHW, _CHW)],
                                 idx_v.at[pl.ds(nbf * _CHW, _CHW)], sem_i),
                pltpu.async_copy(w_ref.at[pl.ds((ck + 1) * _CHW, _CHW)],
                                 w_v.at[pl.ds(nbf * _CHW, _CHW)], sem_w),
            )

        @pl.loop(0, _CH)
        def _node(li, ck=ck, bf=bf):
            i = ck * _CH + li
            row_off = bf * _CHW + li * _FAN_IN
            idxrow = idx_v[pl.ds(row_off, _FAN_IN)]
            wrow = w_v[pl.ds(row_off, _FAN_IN)]
            accs = [None] * _NG
            for j in range(_FAN_IN):
                rj64 = lane_bcast(idxrow, j) * _BPW
                wj = lane_bcast(wrow, j)
                for g in range(_NG):
                    vals = plsc.load_gather(a_ref, [rj64 + cols[g]])
                    t = vals * wj
                    accs[g] = t if accs[g] is None else accs[g] + t
            out_base = (_INPUT_SIZE + i) * _BPW
            for g in range(_NG):
                e = jnp.exp(accs[g] * 2.0)
                y = 1.0 - 2.0 / (e + 1.0)
                a_ref[pl.ds(out_base + _L * g, _L)] = y

    pltpu.sync_copy(
        a_ref.at[pl.ds((_SLOTS - _OUTPUT_SIZE) * _BPW, _OUTPUT_SIZE * _BPW)],
        out_ref.at[wid])


def kernel(x, in_idxs, weights):
    xw = x.reshape(_NW, _BPW, _INPUT_SIZE).transpose(0, 2, 1)
    xw = xw.reshape(_NW, _INPUT_SIZE * _BPW)
    idx = in_idxs.astype(jnp.int32).reshape(-1)
    w_flat = weights.reshape(-1)
    mesh = plsc.VectorSubcoreMesh(core_axis_name="c", subcore_axis_name="s",
                                  num_cores=_NC, num_subcores=_NS)
    outw = pl.kernel(
        _sc_body,
        out_type=jax.ShapeDtypeStruct((_NW, _OUTPUT_SIZE * _BPW), jnp.float32),
        mesh=mesh,
        compiler_params=pltpu.CompilerParams(needs_layout_passes=False),
        scratch_types=[
            pltpu.VMEM((_SLOTS * _BPW,), jnp.float32),
            pltpu.VMEM((2 * _CHW,), jnp.int32),
            pltpu.VMEM((2 * _CHW,), jnp.float32),
            pltpu.SemaphoreType.DMA,
            pltpu.SemaphoreType.DMA,
            pltpu.SemaphoreType.DMA,
        ],
    )(xw, idx, w_flat)
    outw = outw.reshape(_NW, _OUTPUT_SIZE, _BPW)
    return outw.transpose(1, 0, 2).reshape(_OUTPUT_SIZE, _BATCH)


# speedup vs baseline: 1.0433x; 1.0433x over previous
import jax
import jax.numpy as jnp
from jax import lax
from jax.experimental import pallas as pl
from jax.experimental.pallas import tpu as pltpu
from jax.experimental.pallas import tpu_sc as plsc

_NUM_NODES = 1024
_INPUT_SIZE = 512
_OUTPUT_SIZE = 128
_FAN_IN = 16
_BATCH = 2048
_NC = 2
_NS = 16
_L = 16
_NW = _NC * _NS
_BPW = _BATCH // _NW
_NG = _BPW // _L
_CH = 128
_CHW = _CH * _FAN_IN
_NCHK = _NUM_NODES // _CH
_SLOTS = _INPUT_SIZE + _NUM_NODES


def _sc_body(xw_ref, idx_ref, w_ref, out_ref, a_ref, idx_v, w_v,
             sem_x, sem_i, sem_w):
    c = lax.axis_index("c")
    s = lax.axis_index("s")
    wid = s * _NC + c

    cp_x = pltpu.async_copy(xw_ref.at[wid],
                            a_ref.at[pl.ds(0, _INPUT_SIZE * _BPW)], sem_x)
    prev = (
        pltpu.async_copy(idx_ref.at[pl.ds(0, _CHW)],
                         idx_v.at[pl.ds(0, _CHW)], sem_i),
        pltpu.async_copy(w_ref.at[pl.ds(0, _CHW)],
                         w_v.at[pl.ds(0, _CHW)], sem_w),
    )
    cp_x.wait()

    lane = lax.iota(jnp.int32, _L)
    cols = [lane + (_L * g) for g in range(_NG)]
    gdn = lax.GatherDimensionNumbers(
        offset_dims=(), collapsed_slice_dims=(0,), start_index_map=(0,))

    def lane_bcast(vec, j):
        jidx = jnp.full((_L, 1), j, jnp.int32)
        return lax.gather(vec, jidx, gdn, slice_sizes=(1,),
                          mode=lax.GatherScatterMode.PROMISE_IN_BOUNDS)

    for ck in range(_NCHK):
        bf = ck % 2
        prev[0].wait()
        prev[1].wait()
        if ck + 1 < _NCHK:
            nbf = (ck + 1) % 2
            prev = (
                pltpu.async_copy(idx_ref.at[pl.ds((ck + 1) * _CHW, _CHW)],
                                 idx_v.at[pl.ds(nbf * _CHW, _CHW)], sem_i),
                pltpu.async_copy(w_ref.at[pl.ds((ck + 1) * _CHW, _CHW)],
                                 w_v.at[pl.ds(nbf * _CHW, _CHW)], sem_w),
            )

        @pl.loop(0, _CH)
        def _node(li, ck=ck, bf=bf):
            i = ck * _CH + li
            row_off = bf * _CHW + li * _FAN_IN
            idxrow = idx_v[pl.ds(row_off, _FAN_IN)]
            wrow = w_v[pl.ds(row_off, _FAN_IN)]
            accs = [[None, None] for _ in range(_NG)]
            for j in range(_FAN_IN):
                rj64 = lane_bcast(idxrow, j)
                wj = lane_bcast(wrow, j)
                p = j & 1
                for g in range(_NG):
                    vals = plsc.load_gather(a_ref, [rj64 + cols[g]])
                    t = vals * wj
                    accs[g][p] = t if accs[g][p] is None else accs[g][p] + t
            out_base = (_INPUT_SIZE + i) * _BPW
            for g in range(_NG):
                e = jnp.exp((accs[g][0] + accs[g][1]) * 2.0)
                y = 1.0 - 2.0 / (e + 1.0)
                a_ref[pl.ds(out_base + _L * g, _L)] = y

    pltpu.sync_copy(
        a_ref.at[pl.ds((_SLOTS - _OUTPUT_SIZE) * _BPW, _OUTPUT_SIZE * _BPW)],
        out_ref.at[wid])


def kernel(x, in_idxs, weights):
    xw = x.reshape(_NW, _BPW, _INPUT_SIZE).transpose(0, 2, 1)
    xw = xw.reshape(_NW, _INPUT_SIZE * _BPW)
    idx = (in_idxs.astype(jnp.int32) * _BPW).reshape(-1)
    w_flat = weights.reshape(-1)
    mesh = plsc.VectorSubcoreMesh(core_axis_name="c", subcore_axis_name="s",
                                  num_cores=_NC, num_subcores=_NS)
    outw = pl.kernel(
        _sc_body,
        out_type=jax.ShapeDtypeStruct((_NW, _OUTPUT_SIZE * _BPW), jnp.float32),
        mesh=mesh,
        compiler_params=pltpu.CompilerParams(needs_layout_passes=False),
        scratch_types=[
            pltpu.VMEM((_SLOTS * _BPW,), jnp.float32),
            pltpu.VMEM((2 * _CHW,), jnp.int32),
            pltpu.VMEM((2 * _CHW,), jnp.float32),
            pltpu.SemaphoreType.DMA,
            pltpu.SemaphoreType.DMA,
            pltpu.SemaphoreType.DMA,
        ],
    )(xw, idx, w_flat)
    outw = outw.reshape(_NW, _OUTPUT_SIZE, _BPW)
    return outw.transpose(1, 0, 2).reshape(_OUTPUT_SIZE, _BATCH)


# speedup vs baseline: 1.2937x; 1.2399x over previous
import jax
import jax.numpy as jnp
from jax import lax
from jax.experimental import pallas as pl
from jax.experimental.pallas import tpu as pltpu
from jax.experimental.pallas import tpu_sc as plsc

_NUM_NODES = 1024
_INPUT_SIZE = 512
_OUTPUT_SIZE = 128
_FAN_IN = 16
_BATCH = 2048
_SLOTS = _INPUT_SIZE + _NUM_NODES

_TC_B = 1024
_SC_B = _BATCH - _TC_B

_SUB = 8
_LANE = _TC_B // _SUB


def _tc_body(idx_ref, w_ref, x_ref, out_ref, a_ref):
    a_ref[0:_INPUT_SIZE] = x_ref[...]

    def body(i, carry):
        base = i * _FAN_IN
        acc0 = a_ref[pl.ds(idx_ref[base], 1)] * w_ref[base]
        acc1 = a_ref[pl.ds(idx_ref[base + 1], 1)] * w_ref[base + 1]
        for j in range(2, _FAN_IN, 2):
            acc0 = acc0 + a_ref[pl.ds(idx_ref[base + j], 1)] * w_ref[base + j]
            acc1 = (acc1 +
                    a_ref[pl.ds(idx_ref[base + j + 1], 1)] * w_ref[base + j + 1])
        a_ref[pl.ds(_INPUT_SIZE + i, 1)] = jnp.tanh(acc0 + acc1)
        return carry

    jax.lax.fori_loop(0, _NUM_NODES, body, 0)
    out_ref[...] = a_ref[_SLOTS - _OUTPUT_SIZE:]


def _tc_half(x_half, idx_flat, w_flat):
    xT = x_half.T.reshape(_INPUT_SIZE, _SUB, _LANE)
    out = pl.pallas_call(
        _tc_body,
        out_shape=jax.ShapeDtypeStruct((_OUTPUT_SIZE, _SUB, _LANE),
                                       jnp.float32),
        in_specs=[
            pl.BlockSpec(memory_space=pltpu.SMEM),
            pl.BlockSpec(memory_space=pltpu.SMEM),
            pl.BlockSpec(memory_space=pltpu.VMEM),
        ],
        out_specs=pl.BlockSpec(memory_space=pltpu.VMEM),
        scratch_shapes=[pltpu.VMEM((_SLOTS, _SUB, _LANE), jnp.float32)],
    )(idx_flat, w_flat, xT)
    return out.reshape(_OUTPUT_SIZE, _TC_B)


_NC = 2
_NS = 16
_L = 16
_NW = _NC * _NS
_BPW = _SC_B // _NW
_NG = _BPW // _L
_CH = 128
_CHW = _CH * _FAN_IN
_NCHK = _NUM_NODES // _CH


def _sc_body(xw_ref, idx_ref, w_ref, out_ref, a_ref, idx_v, w_v,
             sem_x, sem_i, sem_w):
    c = lax.axis_index("c")
    s = lax.axis_index("s")
    wid = s * _NC + c

    cp_x = pltpu.async_copy(xw_ref.at[wid],
                            a_ref.at[pl.ds(0, _INPUT_SIZE * _BPW)], sem_x)
    prev = (
        pltpu.async_copy(idx_ref.at[pl.ds(0, _CHW)],
                         idx_v.at[pl.ds(0, _CHW)], sem_i),
        pltpu.async_copy(w_ref.at[pl.ds(0, _CHW)],
                         w_v.at[pl.ds(0, _CHW)], sem_w),
    )
    cp_x.wait()

    lane = lax.iota(jnp.int32, _L)
    cols = [lane + (_L * g) for g in range(_NG)]
    gdn = lax.GatherDimensionNumbers(
        offset_dims=(), collapsed_slice_dims=(0,), start_index_map=(0,))

    def lane_bcast(vec, j):
        jidx = jnp.full((_L, 1), j, jnp.int32)
        return lax.gather(vec, jidx, gdn, slice_sizes=(1,),
                          mode=lax.GatherScatterMode.PROMISE_IN_BOUNDS)

    for ck in range(_NCHK):
        bf = ck % 2
        prev[0].wait()
        prev[1].wait()
        if ck + 1 < _NCHK:
            nbf = (ck + 1) % 2
            prev = (
                pltpu.async_copy(idx_ref.at[pl.ds((ck + 1) * _CHW, _CHW)],
                                 idx_v.at[pl.ds(nbf * _CHW, _CHW)], sem_i),
                pltpu.async_copy(w_ref.at[pl.ds((ck + 1) * _CHW, _CHW)],
                                 w_v.at[pl.ds(nbf * _CHW, _CHW)], sem_w),
            )

        @pl.loop(0, _CH)
        def _node(li, ck=ck, bf=bf):
            i = ck * _CH + li
            row_off = bf * _CHW + li * _FAN_IN
            idxrow = idx_v[pl.ds(row_off, _FAN_IN)]
            wrow = w_v[pl.ds(row_off, _FAN_IN)]
            accs = [[None, None] for _ in range(_NG)]
            for j in range(_FAN_IN):
                rj = lane_bcast(idxrow, j)
                wj = lane_bcast(wrow, j)
                p = j & 1
                for g in range(_NG):
                    vals = plsc.load_gather(a_ref, [rj + cols[g]])
                    t = vals * wj
                    accs[g][p] = t if accs[g][p] is None else accs[g][p] + t
            out_base = (_INPUT_SIZE + i) * _BPW
            for g in range(_NG):
                e = jnp.exp((accs[g][0] + accs[g][1]) * 2.0)
                y = 1.0 - 2.0 / (e + 1.0)
                a_ref[pl.ds(out_base + _L * g, _L)] = y

    pltpu.sync_copy(
        a_ref.at[pl.ds((_SLOTS - _OUTPUT_SIZE) * _BPW, _OUTPUT_SIZE * _BPW)],
        out_ref.at[wid])


def _sc_half(x_half, idx, weights):
    xw = x_half.reshape(_NW, _BPW, _INPUT_SIZE).transpose(0, 2, 1)
    xw = xw.reshape(_NW, _INPUT_SIZE * _BPW)
    idx_sc = (idx * _BPW).reshape(-1)
    w_flat = weights.reshape(-1)
    mesh = plsc.VectorSubcoreMesh(core_axis_name="c", subcore_axis_name="s",
                                  num_cores=_NC, num_subcores=_NS)
    outw = pl.kernel(
        _sc_body,
        out_type=jax.ShapeDtypeStruct((_NW, _OUTPUT_SIZE * _BPW), jnp.float32),
        mesh=mesh,
        compiler_params=pltpu.CompilerParams(needs_layout_passes=False),
        scratch_types=[
            pltpu.VMEM((_SLOTS * _BPW,), jnp.float32),
            pltpu.VMEM((2 * _CHW,), jnp.int32),
            pltpu.VMEM((2 * _CHW,), jnp.float32),
            pltpu.SemaphoreType.DMA,
            pltpu.SemaphoreType.DMA,
            pltpu.SemaphoreType.DMA,
        ],
    )(xw, idx_sc, w_flat)
    outw = outw.reshape(_NW, _OUTPUT_SIZE, _BPW)
    return outw.transpose(1, 0, 2).reshape(_OUTPUT_SIZE, _SC_B)


def kernel(x, in_idxs, weights):
    idx = in_idxs.astype(jnp.int32)
    idx_flat = idx.reshape(-1)
    w_flat = weights.reshape(-1)
    out_sc = _sc_half(x[_TC_B:], idx, weights)
    out_tc = _tc_half(x[:_TC_B], idx_flat, w_flat)
    return jnp.concatenate([out_tc, out_sc], axis=1)
